# baseline (speedup 1.0000x reference)
import functools

import jax
import jax.numpy as jnp
from jax import lax
from jax.experimental import pallas as pl
from jax.experimental.pallas import tpu as pltpu
from jax.experimental.pallas import tpu_sc as plsc

_D = 768
_SEQ = 77
_BATCH = 1024
_NC = 2
_NS = 16
_NW = _NC * _NS
_BPW = _BATCH // _NW
_LANES = 16
_PPW = _D // (2 * _LANES)


def _sc_embed(ids_w, tok_w, pos_pk):
  mesh = plsc.VectorSubcoreMesh(core_axis_name="c", subcore_axis_name="s")

  @functools.partial(
      pl.kernel,
      mesh=mesh,
      out_type=jax.ShapeDtypeStruct((_SEQ, _BATCH, _D), jnp.float32),
      scratch_types=[
          pltpu.VMEM((_SEQ * _BPW,), jnp.int32),
          pltpu.VMEM((_SEQ * _PPW * _LANES,), jnp.int32),
          pltpu.VMEM((_BPW, _D), jnp.float32),
          pltpu.VMEM((_BPW, _D), jnp.float32),
          pltpu.VMEM((_BPW, _D), jnp.float32),
          pltpu.SemaphoreType.DMA,
          pltpu.SemaphoreType.DMA,
          pltpu.SemaphoreType.DMA,
          pltpu.SemaphoreType.DMA,
          pltpu.SemaphoreType.DMA,
          pltpu.SemaphoreType.DMA,
      ],
  )
  def k(ids_hbm, tab_hbm, pos_hbm, out_hbm, idx_v, pos_v, buf_a, buf_b,
        buf_c, gsem_a, gsem_b, gsem_c, wsem_a, wsem_b, wsem_c):
    wid = lax.axis_index("s") * _NC + lax.axis_index("c")
    b0 = wid * _BPW
    pltpu.sync_copy(ids_hbm.at[pl.ds(wid * _SEQ * _BPW, _SEQ * _BPW)], idx_v)
    pltpu.sync_copy(pos_hbm, pos_v)

    def g_start(s, buf, gsem):
      pltpu.async_copy(tab_hbm.at[idx_v.at[pl.ds(s * _BPW, _BPW)]], buf, gsem)

    def g_wait(s, buf, gsem):
      pltpu.make_async_copy(
          tab_hbm.at[idx_v.at[pl.ds(s * _BPW, _BPW)]], buf, gsem).wait()

    def w_start(s, buf, wsem):
      pltpu.async_copy(buf, out_hbm.at[s, pl.ds(b0, _BPW)], wsem)

    def w_wait(s, buf, wsem):
      pltpu.make_async_copy(buf, out_hbm.at[s, pl.ds(b0, _BPW)], wsem).wait()

    def add_pos(s, buf):
      pchunks = []
      for cp in range(_PPW):
        packed = pos_v[pl.ds(s * (_D // 2) + _LANES * cp, _LANES)]
        pchunks.append(lax.bitcast_convert_type(packed << 16, jnp.float32))
        pchunks.append(
            lax.bitcast_convert_type(packed & jnp.int32(-65536), jnp.float32))

      @plsc.parallel_loop(0, _BPW)
      def _(i):
        for c in range(_D // _LANES):
          sl = pl.ds(c * _LANES, _LANES)
          buf[i, sl] = buf[i, sl] + pchunks[c]

    bufs = (buf_a, buf_b, buf_c)
    gsems = (gsem_a, gsem_b, gsem_c)
    wsems = (wsem_a, wsem_b, wsem_c)

    g_start(0, buf_a, gsem_a)
    g_start(1, buf_b, gsem_b)

    def window(s, r, first=False):
      g_wait(s, bufs[r], gsems[r])
      add_pos(s, bufs[r])
      w_start(s, bufs[r], wsems[r])
      rn = (r + 2) % 3
      if not first:
        w_wait(s - 1, bufs[rn], wsems[rn])
      g_start(s + 2, bufs[rn], gsems[rn])

    ntri = (_SEQ - 2) // 3

    def tri_body(t, carry):
      s0 = 3 * t
      window(s0, 0, first=False)
      window(s0 + 1, 1)
      window(s0 + 2, 2)
      return carry

    window(0, 0, first=True)
    window(1, 1)
    window(2, 2)
    lax.fori_loop(1, ntri, tri_body, 0)

    for s, r in ((_SEQ - 2, 0), (_SEQ - 1, 1)):
      g_wait(s, bufs[r], gsems[r])
      add_pos(s, bufs[r])
      w_start(s, bufs[r], wsems[r])
      rn = (r + 2) % 3
      w_wait(s - 1, bufs[rn], wsems[rn])
    w_wait(_SEQ - 1, buf_b, wsem_b)

  return k(ids_w, tok_w, pos_pk)


def kernel(input_ids, token_embedding_weight, position_embedding_weight):
  ids = input_ids.astype(jnp.int32)
  ids_w = ids.T.reshape(_SEQ, _NW, _BPW).transpose(1, 0, 2).reshape(-1)
  bits = lax.bitcast_convert_type(
      position_embedding_weight.astype(jnp.bfloat16), jnp.uint16
  ).reshape(-1, 2, _LANES).astype(jnp.uint32)
  pos_pk = lax.bitcast_convert_type(
      bits[:, 0, :] | (bits[:, 1, :] << 16), jnp.int32).reshape(-1)
  out_t = _sc_embed(ids_w, token_embedding_weight, pos_pk)
  return out_t.transpose(1, 0, 2)


# speedup vs baseline: 1.0145x; 1.0145x over previous
import functools

import jax
import jax.numpy as jnp
from jax import lax
from jax.experimental import pallas as pl
from jax.experimental.pallas import tpu as pltpu
from jax.experimental.pallas import tpu_sc as plsc

_D = 768
_SEQ = 77
_BATCH = 1024
_NC = 2
_NS = 16
_NW = _NC * _NS
_BPW = _BATCH // _NW
_LANES = 16
_PPW = _D // (2 * _LANES)


def _sc_embed(ids_w, tok_w, pos_pk):
  mesh = plsc.VectorSubcoreMesh(core_axis_name="c", subcore_axis_name="s")

  @functools.partial(
      pl.kernel,
      mesh=mesh,
      out_type=jax.ShapeDtypeStruct((_SEQ, _BATCH, _D), jnp.float32),
      scratch_types=[
          pltpu.VMEM((_SEQ * _BPW,), jnp.int32),
          pltpu.VMEM((_SEQ * _D,), jnp.float32),
          pltpu.VMEM((_BPW, _D), jnp.float32),
          pltpu.VMEM((_BPW, _D), jnp.float32),
          pltpu.SemaphoreType.DMA,
          pltpu.SemaphoreType.DMA,
          pltpu.SemaphoreType.DMA,
          pltpu.SemaphoreType.DMA,
      ],
  )
  def k(ids_hbm, tab_hbm, pos_hbm, out_hbm, idx_v, pos_v, buf_a, buf_b,
        gsem_a, gsem_b, wsem_a, wsem_b):
    wid = lax.axis_index("s") * _NC + lax.axis_index("c")
    b0 = wid * _BPW
    pltpu.sync_copy(ids_hbm.at[pl.ds(wid * _SEQ * _BPW, _SEQ * _BPW)], idx_v)
    pltpu.sync_copy(pos_hbm, pos_v)

    def g_start(s, buf, gsem):
      pltpu.async_copy(tab_hbm.at[idx_v.at[pl.ds(s * _BPW, _BPW)]], buf, gsem)

    def g_wait(s, buf, gsem):
      pltpu.make_async_copy(
          tab_hbm.at[idx_v.at[pl.ds(s * _BPW, _BPW)]], buf, gsem).wait()

    def w_start(s, buf, wsem):
      pltpu.async_copy(buf, out_hbm.at[s, pl.ds(b0, _BPW)], wsem)

    def w_wait(s, buf, wsem):
      pltpu.make_async_copy(buf, out_hbm.at[s, pl.ds(b0, _BPW)], wsem).wait()

    def add_pos(s, buf):
      pchunks = [
          pos_v[pl.ds(s * _D + c * _LANES, _LANES)]
          for c in range(_D // _LANES)
      ]

      @plsc.parallel_loop(0, _BPW)
      def _(i):
        for c in range(_D // _LANES):
          sl = pl.ds(c * _LANES, _LANES)
          buf[i, sl] = buf[i, sl] + pchunks[c]

    g_start(0, buf_a, gsem_a)
    g_start(1, buf_b, gsem_b)

    npair = _SEQ // 2

    def pair_body(t, carry):
      s = 2 * t
      g_wait(s, buf_a, gsem_a)
      add_pos(s, buf_a)
      w_start(s, buf_a, wsem_a)
      g_wait(s + 1, buf_b, gsem_b)
      add_pos(s + 1, buf_b)
      w_start(s + 1, buf_b, wsem_b)

      @pl.when(s + 2 < _SEQ)
      def _():
        w_wait(s, buf_a, wsem_a)
        g_start(s + 2, buf_a, gsem_a)

      @pl.when(s + 3 < _SEQ)
      def _():
        w_wait(s + 1, buf_b, wsem_b)
        g_start(s + 3, buf_b, gsem_b)

      return carry

    lax.fori_loop(0, npair, pair_body, 0)

    s_last = _SEQ - 1
    g_wait(s_last, buf_a, gsem_a)
    add_pos(s_last, buf_a)
    w_start(s_last, buf_a, wsem_a)
    w_wait(s_last, buf_a, wsem_a)
    w_wait(s_last - 1, buf_b, wsem_b)

  return k(ids_w, tok_w, pos_pk)


def kernel(input_ids, token_embedding_weight, position_embedding_weight):
  ids = input_ids.astype(jnp.int32)
  ids_w = ids.T.reshape(_SEQ, _NW, _BPW).transpose(1, 0, 2).reshape(-1)
  pos_flat = position_embedding_weight.reshape(-1)
  out_t = _sc_embed(ids_w, token_embedding_weight, pos_flat)
  return out_t.transpose(1, 0, 2)
